# Initial kernel scaffold; baseline (speedup 1.0000x reference)
import jax
import jax.numpy as jnp
from jax.experimental import pallas as pl


def kernel(x, edge_index, W1, beta1, W2, beta2):
    raise NotImplementedError("write your pallas kernel here")


# speedup vs baseline: 12.7596x; 12.7596x over previous
import jax
import jax.numpy as jnp
from jax import lax
from jax.experimental import pallas as pl
from jax.experimental.pallas import tpu as pltpu
from jax.experimental.pallas import tpu_sc as plsc

N = 10000
E = 320000
D = 128
NPAD = 10240
ROWS_PER_TILE = NPAD // 16
CH = 80
EPT = E // 32
NCH = EPT // CH
RB = 2048
ZR = 8192 // D


def _rsqrt_nr(x):
    i = plsc.bitcast(x, jnp.int32)
    i = jnp.int32(0x5F3759DF) - lax.shift_right_logical(i, 1)
    y = plsc.bitcast(i, jnp.float32)
    half = x * 0.5
    for _ in range(3):
        y = y * (1.5 - half * y * y)
    return y


def _sc_edge_layer(tab, src, dst, beta16):
    def body(tab_h, src_h, dst_h, beta_h, outp_h, outd_h,
             src_v, dst_v, S_v, T_v, R_v, ex_v,
             zb_v, zd_v, bv, acc_s, den_s, sem1, sem2):
        cid = lax.axis_index("c")
        sid = lax.axis_index("s")

        pltpu.sync_copy(beta_h, bv)

        def zb_row(i, _):
            for j in range(D // 16):
                zb_v[i, pl.ds(j * 16, 16)] = jnp.zeros((16,), jnp.float32)
            return _
        lax.fori_loop(0, ZR, zb_row, 0)

        def zd_row(i, _):
            zd_v[pl.ds(i * 16, 16)] = jnp.zeros((16,), jnp.float32)
            return _
        lax.fori_loop(0, ROWS_PER_TILE // 16, zd_row, 0)

        for j in range(ROWS_PER_TILE // ZR):
            pltpu.sync_copy(zb_v, acc_s.at[pl.ds(sid * ROWS_PER_TILE + j * ZR, ZR)])
        pltpu.sync_copy(zd_v, den_s.at[pl.ds(sid * ROWS_PER_TILE, ROWS_PER_TILE)])

        plsc.subcore_barrier()

        bvec = bv[...]
        base = cid * (16 * EPT) + sid * EPT
        iota16 = lax.broadcasted_iota(jnp.int32, (16,), 0)

        def chunk(k, _):
            off = base + k * CH
            pltpu.sync_copy(src_h.at[pl.ds(off, CH)], src_v)
            pltpu.sync_copy(dst_h.at[pl.ds(off, CH)], dst_v)
            c1 = pltpu.async_copy(tab_h.at[src_v], S_v, sem1)
            c2 = pltpu.async_copy(tab_h.at[dst_v], T_v, sem2)
            c1.wait()
            c2.wait()

            def group(g, _):
                sl = pl.ds(g * 16, 16)
                st = jnp.zeros((16,), jnp.float32)
                ss = jnp.zeros((16,), jnp.float32)
                tt = jnp.zeros((16,), jnp.float32)
                for l in range(16):
                    e = g * 16 + l
                    s0 = S_v[e, pl.ds(0, 16)]
                    t0 = T_v[e, pl.ds(0, 16)]
                    a = s0 * t0
                    b = s0 * s0
                    c = t0 * t0
                    for j in range(1, D // 16):
                        dsl = pl.ds(j * 16, 16)
                        sj = S_v[e, dsl]
                        tj = T_v[e, dsl]
                        a = a + sj * tj
                        b = b + sj * sj
                        c = c + tj * tj
                    lm = iota16 == l
                    st = jnp.where(lm, jnp.sum(a), st)
                    ss = jnp.where(lm, jnp.sum(b), ss)
                    tt = jnp.where(lm, jnp.sum(c), tt)
                cos = st * _rsqrt_nr(jnp.maximum(ss * tt, 1e-30))
                ex = jnp.exp(cos * bvec)
                ex_v[sl] = ex
                for l in range(16):
                    e = g * 16 + l
                    mm = ex[l]
                    for j in range(D // 16):
                        dsl = pl.ds(j * 16, 16)
                        R_v[e, dsl] = S_v[e, dsl] * mm
                return _
            lax.fori_loop(0, CH // 16, group, 0)

            pltpu.sync_copy(ex_v, den_s.at[dst_v], add=True)
            pltpu.sync_copy(R_v, acc_s.at[dst_v], add=True)
            return _
        lax.fori_loop(0, NCH, chunk, 0)

        plsc.subcore_barrier()

        for j in range(ROWS_PER_TILE // CH):
            r0 = sid * ROWS_PER_TILE + j * CH
            pltpu.sync_copy(acc_s.at[pl.ds(r0, CH)], S_v)
            pltpu.sync_copy(S_v, outp_h.at[cid, pl.ds(r0, CH)])
        for j in range(ROWS_PER_TILE // CH):
            r0 = sid * ROWS_PER_TILE + j * CH
            pltpu.sync_copy(den_s.at[pl.ds(r0, CH)], ex_v)
            pltpu.sync_copy(ex_v, outd_h.at[pl.ds(cid * NPAD + r0, CH)])

    mesh = plsc.VectorSubcoreMesh(core_axis_name="c", subcore_axis_name="s")
    f = pl.kernel(
        body,
        out_type=[
            jax.ShapeDtypeStruct((2, NPAD, D), jnp.float32),
            jax.ShapeDtypeStruct((2 * NPAD,), jnp.float32),
        ],
        mesh=mesh,
        compiler_params=pltpu.CompilerParams(needs_layout_passes=False),
        scratch_types=[
            pltpu.VMEM((CH,), jnp.int32),
            pltpu.VMEM((CH,), jnp.int32),
            pltpu.VMEM((CH, D), jnp.float32),
            pltpu.VMEM((CH, D), jnp.float32),
            pltpu.VMEM((CH, D), jnp.float32),
            pltpu.VMEM((CH,), jnp.float32),
            pltpu.VMEM((ZR, D), jnp.float32),
            pltpu.VMEM((ROWS_PER_TILE,), jnp.float32),
            pltpu.VMEM((16,), jnp.float32),
            pltpu.VMEM_SHARED((NPAD, D), jnp.float32),
            pltpu.VMEM_SHARED((NPAD,), jnp.float32),
            pltpu.SemaphoreType.DMA,
            pltpu.SemaphoreType.DMA,
        ],
    )
    acc, den = f(tab, src, dst, beta16)
    return acc, den.reshape(2, NPAD)


def _tc_proj(x, W):
    Din = W.shape[0]

    def tc_body(x_ref, w_ref, o_ref):
        o_ref[...] = jnp.dot(x_ref[...], w_ref[...],
                             preferred_element_type=jnp.float32)

    return pl.pallas_call(
        tc_body,
        grid=(NPAD // RB,),
        in_specs=[
            pl.BlockSpec((RB, Din), lambda i: (i, 0)),
            pl.BlockSpec((Din, D), lambda i: (0, 0)),
        ],
        out_specs=pl.BlockSpec((RB, D), lambda i: (i, 0)),
        out_shape=jax.ShapeDtypeStruct((NPAD, D), jnp.float32),
    )(x, W)


def _tc_combine_proj(p, dnm, W):
    C = W.shape[1]

    def tc_body(p0_ref, p1_ref, d0_ref, d1_ref, w_ref, o_ref):
        den = d0_ref[...] + d1_ref[...] + 1e-16
        z = jnp.maximum((p0_ref[...] + p1_ref[...]) / den[:, None], 0.0)
        h = jnp.dot(z, w_ref[...], preferred_element_type=jnp.float32)
        o_ref[...] = jnp.concatenate(
            [h, jnp.zeros((RB, D - C), jnp.float32)], axis=1)

    return pl.pallas_call(
        tc_body,
        grid=(NPAD // RB,),
        in_specs=[
            pl.BlockSpec((RB, D), lambda i: (i, 0)),
            pl.BlockSpec((RB, D), lambda i: (i, 0)),
            pl.BlockSpec((RB,), lambda i: (i,)),
            pl.BlockSpec((RB,), lambda i: (i,)),
            pl.BlockSpec((D, C), lambda i: (0, 0)),
        ],
        out_specs=pl.BlockSpec((RB, D), lambda i: (i, 0)),
        out_shape=jax.ShapeDtypeStruct((NPAD, D), jnp.float32),
    )(p[0], p[1], dnm[0], dnm[1], W)


def _tc_combine_logsoftmax(q, dnm, C):
    def tc_body(q0_ref, q1_ref, d0_ref, d1_ref, o_ref):
        den = d0_ref[...] + d1_ref[...] + 1e-16
        o = (q0_ref[...] + q1_ref[...])[:, :C] / den[:, None]
        m = jnp.max(o, axis=1, keepdims=True)
        ex = jnp.exp(o - m)
        lse = jnp.log(jnp.sum(ex, axis=1, keepdims=True))
        o_ref[...] = o - m - lse

    return pl.pallas_call(
        tc_body,
        grid=(NPAD // RB,),
        in_specs=[
            pl.BlockSpec((RB, D), lambda i: (i, 0)),
            pl.BlockSpec((RB, D), lambda i: (i, 0)),
            pl.BlockSpec((RB,), lambda i: (i,)),
            pl.BlockSpec((RB,), lambda i: (i,)),
        ],
        out_specs=pl.BlockSpec((RB, C), lambda i: (i, 0)),
        out_shape=jax.ShapeDtypeStruct((NPAD, C), jnp.float32),
    )(q[0], q[1], dnm[0], dnm[1])


@jax.jit
def kernel(x, edge_index, W1, beta1, W2, beta2):
    src = edge_index[0]
    dst = edge_index[1]
    x_pad = jnp.pad(x, ((0, NPAD - N), (0, 0)))

    tab1 = _tc_proj(x_pad, W1)
    b1 = jnp.full((16,), beta1, jnp.float32)
    p1, d1 = _sc_edge_layer(tab1, src, dst, b1)

    tab2 = _tc_combine_proj(p1, d1, W2)
    b2 = jnp.full((16,), beta2, jnp.float32)
    p2, d2 = _sc_edge_layer(tab2, src, dst, b2)

    out = _tc_combine_logsoftmax(p2, d2, W2.shape[1])
    return out[:N]


# speedup vs baseline: 24.2578x; 1.9011x over previous
import jax
import jax.numpy as jnp
from jax import lax
from jax.experimental import pallas as pl
from jax.experimental.pallas import tpu as pltpu
from jax.experimental.pallas import tpu_sc as plsc

N = 10000
E = 320000
D = 128
NPAD = 10240
ROWS_PER_TILE = NPAD // 16
CH = 80
EPT = E // 32
NCH = EPT // CH
RB = 2048


def _rsqrt_nr(x):
    i = plsc.bitcast(x, jnp.int32)
    i = jnp.int32(0x5F3759DF) - lax.shift_right_logical(i, 1)
    y = plsc.bitcast(i, jnp.float32)
    half = x * 0.5
    for _ in range(3):
        y = y * (1.5 - half * y * y)
    return y


def _sc_edge_layer(tab, src, dst, beta16):
    def body(tab_h, src_h, dst_h, beta_h, outp_h, outd_h,
             sc0_v, dc0_v, sc1_v, dc1_v, dscat_v,
             S0_v, T0_v, S1_v, T1_v, ex_v, bv,
             acc_s, den_s, si0, si1, gsS0, gsT0, gsS1, gsT1):
        cid = lax.axis_index("c")
        sid = lax.axis_index("s")
        ebase = (cid * 16 + sid) * EPT

        pltpu.sync_copy(beta_h, bv)

        def idx_issue(k, sc_v, dc_v, sem):
            off = ebase + k * CH
            pltpu.async_copy(src_h.at[pl.ds(off, CH)], sc_v, sem)
            pltpu.async_copy(dst_h.at[pl.ds(off, CH)], dc_v, sem)

        def idx_wait(k, sc_v, dc_v, sem):
            off = ebase + k * CH
            pltpu.make_async_copy(src_h.at[pl.ds(off, CH)], sc_v, sem).wait()
            pltpu.make_async_copy(dst_h.at[pl.ds(off, CH)], dc_v, sem).wait()

        idx_issue(0, sc0_v, dc0_v, si0)
        idx_issue(1, sc1_v, dc1_v, si1)

        def zs_row(i, _):
            for j in range(D // 16):
                S0_v[i, pl.ds(j * 16, 16)] = jnp.zeros((16,), jnp.float32)
            return _
        lax.fori_loop(0, CH, zs_row, 0)
        for g in range(CH // 16):
            ex_v[pl.ds(g * 16, 16)] = jnp.zeros((16,), jnp.float32)

        for j in range(ROWS_PER_TILE // CH):
            r0 = sid * ROWS_PER_TILE + j * CH
            pltpu.sync_copy(S0_v, acc_s.at[pl.ds(r0, CH)])
            pltpu.sync_copy(ex_v, den_s.at[pl.ds(r0, CH)])

        plsc.subcore_barrier()

        bvec = bv[...]
        iota16 = lax.broadcasted_iota(jnp.int32, (16,), 0)

        def compute(S_v, T_v):
            def group(g, _):
                sl = pl.ds(g * 16, 16)
                st = jnp.zeros((16,), jnp.float32)
                ss = jnp.zeros((16,), jnp.float32)
                tt = jnp.zeros((16,), jnp.float32)
                for l in range(16):
                    e = g * 16 + l
                    s0 = S_v[e, pl.ds(0, 16)]
                    t0 = T_v[e, pl.ds(0, 16)]
                    a = s0 * t0
                    b = s0 * s0
                    c = t0 * t0
                    for j in range(1, D // 16):
                        dsl = pl.ds(j * 16, 16)
                        sj = S_v[e, dsl]
                        tj = T_v[e, dsl]
                        a = a + sj * tj
                        b = b + sj * sj
                        c = c + tj * tj
                    lm = iota16 == l
                    st = jnp.where(lm, jnp.sum(a), st)
                    ss = jnp.where(lm, jnp.sum(b), ss)
                    tt = jnp.where(lm, jnp.sum(c), tt)
                cos = st * _rsqrt_nr(jnp.maximum(ss * tt, 1e-30))
                ex = jnp.exp(cos * bvec)
                ex_v[sl] = ex
                for l in range(16):
                    e = g * 16 + l
                    mm = ex[l]
                    for j in range(D // 16):
                        dsl = pl.ds(j * 16, 16)
                        S_v[e, dsl] = S_v[e, dsl] * mm
                return _
            lax.fori_loop(0, CH // 16, group, 0)

            pltpu.sync_copy(ex_v, den_s.at[dscat_v], add=True)
            pltpu.sync_copy(S_v, acc_s.at[dscat_v], add=True)

        def rows_issue(S_v, T_v, sc_v, dc_v, sS, sT):
            pltpu.async_copy(tab_h.at[sc_v], S_v, sS)
            pltpu.async_copy(tab_h.at[dc_v], T_v, sT)

        def rows_wait(S_v, T_v, sc_v, dc_v, sS, sT):
            pltpu.make_async_copy(tab_h.at[sc_v], S_v, sS).wait()
            pltpu.make_async_copy(tab_h.at[dc_v], T_v, sT).wait()

        bufs = [(S0_v, T0_v, sc0_v, dc0_v, si0, gsS0, gsT0),
                (S1_v, T1_v, sc1_v, dc1_v, si1, gsS1, gsT1)]

        def step(k, b):
            S_v, T_v, sc_v, dc_v, si, sS, sT = bufs[b]
            Sn, Tn, scn, dcn, sin, sSn, sTn = bufs[1 - b]
            rows_wait(S_v, T_v, sc_v, dc_v, sS, sT)
            for g in range(CH // 16):
                sl16 = pl.ds(g * 16, 16)
                dscat_v[sl16] = dc_v[sl16]
            idx_issue(k + 2, sc_v, dc_v, si)
            idx_wait(k + 1, scn, dcn, sin)
            rows_issue(Sn, Tn, scn, dcn, sSn, sTn)
            compute(S_v, T_v)

        idx_wait(0, sc0_v, dc0_v, si0)
        rows_issue(S0_v, T0_v, sc0_v, dc0_v, gsS0, gsT0)

        def pair(p, _):
            step(2 * p, 0)
            step(2 * p + 1, 1)
            return _
        lax.fori_loop(0, (NCH - 1) // 2, pair, 0)

        rows_wait(S0_v, T0_v, sc0_v, dc0_v, gsS0, gsT0)
        for g in range(CH // 16):
            sl16 = pl.ds(g * 16, 16)
            dscat_v[sl16] = dc0_v[sl16]
        compute(S0_v, T0_v)
        idx_wait(NCH, sc1_v, dc1_v, si1)

        plsc.subcore_barrier()

        for j in range(ROWS_PER_TILE // CH):
            r0 = sid * ROWS_PER_TILE + j * CH
            pltpu.sync_copy(acc_s.at[pl.ds(r0, CH)], S0_v)
            pltpu.sync_copy(S0_v, outp_h.at[cid, pl.ds(r0, CH)])
        for j in range(ROWS_PER_TILE // CH):
            r0 = sid * ROWS_PER_TILE + j * CH
            pltpu.sync_copy(den_s.at[pl.ds(r0, CH)], ex_v)
            pltpu.sync_copy(ex_v, outd_h.at[pl.ds(cid * NPAD + r0, CH)])

    mesh = plsc.VectorSubcoreMesh(core_axis_name="c", subcore_axis_name="s")
    f = pl.kernel(
        body,
        out_type=[
            jax.ShapeDtypeStruct((2, NPAD, D), jnp.float32),
            jax.ShapeDtypeStruct((2 * NPAD,), jnp.float32),
        ],
        mesh=mesh,
        compiler_params=pltpu.CompilerParams(needs_layout_passes=False),
        scratch_types=[
            pltpu.VMEM((CH,), jnp.int32),
            pltpu.VMEM((CH,), jnp.int32),
            pltpu.VMEM((CH,), jnp.int32),
            pltpu.VMEM((CH,), jnp.int32),
            pltpu.VMEM((CH,), jnp.int32),
            pltpu.VMEM((CH, D), jnp.float32),
            pltpu.VMEM((CH, D), jnp.float32),
            pltpu.VMEM((CH, D), jnp.float32),
            pltpu.VMEM((CH, D), jnp.float32),
            pltpu.VMEM((CH,), jnp.float32),
            pltpu.VMEM((16,), jnp.float32),
            pltpu.VMEM_SHARED((NPAD, D), jnp.float32),
            pltpu.VMEM_SHARED((NPAD,), jnp.float32),
            pltpu.SemaphoreType.DMA,
            pltpu.SemaphoreType.DMA,
            pltpu.SemaphoreType.DMA,
            pltpu.SemaphoreType.DMA,
            pltpu.SemaphoreType.DMA,
            pltpu.SemaphoreType.DMA,
        ],
    )
    acc, den = f(tab, src, dst, beta16)
    return acc, den.reshape(2, NPAD)


def _tc_proj(x, W):
    Din = W.shape[0]

    def tc_body(x_ref, w_ref, o_ref):
        o_ref[...] = jnp.dot(x_ref[...], w_ref[...],
                             preferred_element_type=jnp.float32)

    return pl.pallas_call(
        tc_body,
        grid=(NPAD // RB,),
        in_specs=[
            pl.BlockSpec((RB, Din), lambda i: (i, 0)),
            pl.BlockSpec((Din, D), lambda i: (0, 0)),
        ],
        out_specs=pl.BlockSpec((RB, D), lambda i: (i, 0)),
        out_shape=jax.ShapeDtypeStruct((NPAD, D), jnp.float32),
    )(x, W)


def _tc_combine_proj(p, dnm, W):
    C = W.shape[1]

    def tc_body(p0_ref, p1_ref, d0_ref, d1_ref, w_ref, o_ref):
        den = d0_ref[...] + d1_ref[...] + 1e-16
        z = jnp.maximum((p0_ref[...] + p1_ref[...]) / den[:, None], 0.0)
        h = jnp.dot(z, w_ref[...], preferred_element_type=jnp.float32)
        o_ref[...] = jnp.concatenate(
            [h, jnp.zeros((RB, D - C), jnp.float32)], axis=1)

    return pl.pallas_call(
        tc_body,
        grid=(NPAD // RB,),
        in_specs=[
            pl.BlockSpec((RB, D), lambda i: (i, 0)),
            pl.BlockSpec((RB, D), lambda i: (i, 0)),
            pl.BlockSpec((RB,), lambda i: (i,)),
            pl.BlockSpec((RB,), lambda i: (i,)),
            pl.BlockSpec((D, C), lambda i: (0, 0)),
        ],
        out_specs=pl.BlockSpec((RB, D), lambda i: (i, 0)),
        out_shape=jax.ShapeDtypeStruct((NPAD, D), jnp.float32),
    )(p[0], p[1], dnm[0], dnm[1], W)


def _tc_combine_logsoftmax(q, dnm, C):
    def tc_body(q0_ref, q1_ref, d0_ref, d1_ref, o_ref):
        den = d0_ref[...] + d1_ref[...] + 1e-16
        o = (q0_ref[...] + q1_ref[...])[:, :C] / den[:, None]
        m = jnp.max(o, axis=1, keepdims=True)
        ex = jnp.exp(o - m)
        lse = jnp.log(jnp.sum(ex, axis=1, keepdims=True))
        o_ref[...] = o - m - lse

    return pl.pallas_call(
        tc_body,
        grid=(NPAD // RB,),
        in_specs=[
            pl.BlockSpec((RB, D), lambda i: (i, 0)),
            pl.BlockSpec((RB, D), lambda i: (i, 0)),
            pl.BlockSpec((RB,), lambda i: (i,)),
            pl.BlockSpec((RB,), lambda i: (i,)),
        ],
        out_specs=pl.BlockSpec((RB, C), lambda i: (i, 0)),
        out_shape=jax.ShapeDtypeStruct((NPAD, C), jnp.float32),
    )(q[0], q[1], dnm[0], dnm[1])


@jax.jit
def kernel(x, edge_index, W1, beta1, W2, beta2):
    src = jnp.pad(edge_index[0], (0, 2 * CH))
    dst = jnp.pad(edge_index[1], (0, 2 * CH))
    x_pad = jnp.pad(x, ((0, NPAD - N), (0, 0)))

    tab1 = _tc_proj(x_pad, W1)
    b1 = jnp.full((16,), beta1, jnp.float32)
    p1, d1 = _sc_edge_layer(tab1, src, dst, b1)

    tab2 = _tc_combine_proj(p1, d1, W2)
    b2 = jnp.full((16,), beta2, jnp.float32)
    p2, d2 = _sc_edge_layer(tab2, src, dst, b2)

    out = _tc_combine_logsoftmax(p2, d2, W2.shape[1])
    return out[:N]


# speedup vs baseline: 25.0864x; 1.0342x over previous
import jax
import jax.numpy as jnp
from jax import lax
from jax.experimental import pallas as pl
from jax.experimental.pallas import tpu as pltpu
from jax.experimental.pallas import tpu_sc as plsc

N = 10000
E = 320000
D = 128
NPAD = 10240
ROWS_PER_TILE = NPAD // 16
CH = 80
EPT = E // 32
NCH = EPT // CH
RB = 2048


def _rsqrt_nr(x):
    i = plsc.bitcast(x, jnp.int32)
    i = jnp.int32(0x5F3759DF) - lax.shift_right_logical(i, 1)
    y = plsc.bitcast(i, jnp.float32)
    half = x * 0.5
    for _ in range(3):
        y = y * (1.5 - half * y * y)
    return y


def _sc_edge_layer(tab, src, dst, beta16):
    def body(tab_h, src_h, dst_h, beta_h, outp_h, outd_h,
             sc0_v, dc0_v, sc1_v, dc1_v, ds0_v, ds1_v,
             S0_v, T0_v, S1_v, T1_v, ex0_v, ex1_v, bv,
             acc_s, den_s, si0, si1, gsS0, gsT0, gsS1, gsT1, ss0, ss1):
        cid = lax.axis_index("c")
        sid = lax.axis_index("s")
        ebase = (cid * 16 + sid) * EPT

        pltpu.sync_copy(beta_h, bv)

        def idx_issue(k, sc_v, dc_v, sem):
            off = ebase + k * CH
            pltpu.async_copy(src_h.at[pl.ds(off, CH)], sc_v, sem)
            pltpu.async_copy(dst_h.at[pl.ds(off, CH)], dc_v, sem)

        def idx_wait(k, sc_v, dc_v, sem):
            off = ebase + k * CH
            pltpu.make_async_copy(src_h.at[pl.ds(off, CH)], sc_v, sem).wait()
            pltpu.make_async_copy(dst_h.at[pl.ds(off, CH)], dc_v, sem).wait()

        idx_issue(0, sc0_v, dc0_v, si0)
        idx_issue(1, sc1_v, dc1_v, si1)

        def zs_row(i, _):
            for j in range(D // 16):
                S0_v[i, pl.ds(j * 16, 16)] = jnp.zeros((16,), jnp.float32)
            return _
        lax.fori_loop(0, CH, zs_row, 0)
        for g in range(CH // 16):
            ex0_v[pl.ds(g * 16, 16)] = jnp.zeros((16,), jnp.float32)

        for j in range(ROWS_PER_TILE // CH):
            r0 = sid * ROWS_PER_TILE + j * CH
            pltpu.sync_copy(S0_v, acc_s.at[pl.ds(r0, CH)])
            pltpu.sync_copy(ex0_v, den_s.at[pl.ds(r0, CH)])

        plsc.subcore_barrier()

        bvec = bv[...]
        iota16 = lax.broadcasted_iota(jnp.int32, (16,), 0)

        def compute(S_v, T_v, ex_v, dscat_v, ssem):
            def group(g, _):
                sl = pl.ds(g * 16, 16)
                st = jnp.zeros((16,), jnp.float32)
                ss = jnp.zeros((16,), jnp.float32)
                tt = jnp.zeros((16,), jnp.float32)
                for l in range(16):
                    e = g * 16 + l
                    s0 = S_v[e, pl.ds(0, 16)]
                    t0 = T_v[e, pl.ds(0, 16)]
                    a = s0 * t0
                    b = s0 * s0
                    c = t0 * t0
                    for j in range(1, D // 16):
                        dsl = pl.ds(j * 16, 16)
                        sj = S_v[e, dsl]
                        tj = T_v[e, dsl]
                        a = a + sj * tj
                        b = b + sj * sj
                        c = c + tj * tj
                    lm = iota16 == l
                    st = jnp.where(lm, jnp.sum(a), st)
                    ss = jnp.where(lm, jnp.sum(b), ss)
                    tt = jnp.where(lm, jnp.sum(c), tt)
                cos = st * _rsqrt_nr(jnp.maximum(ss * tt, 1e-30))
                ex = jnp.exp(cos * bvec)
                ex_v[sl] = ex
                for l in range(16):
                    e = g * 16 + l
                    mm = ex[l]
                    for j in range(D // 16):
                        dsl = pl.ds(j * 16, 16)
                        S_v[e, dsl] = S_v[e, dsl] * mm
                return _
            lax.fori_loop(0, CH // 16, group, 0)

            pltpu.async_copy(ex_v, den_s.at[dscat_v], ssem, add=True)
            pltpu.async_copy(S_v, acc_s.at[dscat_v], ssem, add=True)

        def scat_wait(S_v, ex_v, dscat_v, ssem):
            pltpu.make_async_copy(ex_v, den_s.at[dscat_v], ssem).wait()
            pltpu.make_async_copy(S_v, acc_s.at[dscat_v], ssem).wait()

        def rows_issue(S_v, T_v, sc_v, dc_v, sS, sT):
            pltpu.async_copy(tab_h.at[sc_v], S_v, sS)
            pltpu.async_copy(tab_h.at[dc_v], T_v, sT)

        def rows_wait(S_v, T_v, sc_v, dc_v, sS, sT):
            pltpu.make_async_copy(tab_h.at[sc_v], S_v, sS).wait()
            pltpu.make_async_copy(tab_h.at[dc_v], T_v, sT).wait()

        bufs = [(S0_v, T0_v, sc0_v, dc0_v, si0, gsS0, gsT0, ex0_v, ds0_v, ss0),
                (S1_v, T1_v, sc1_v, dc1_v, si1, gsS1, gsT1, ex1_v, ds1_v, ss1)]

        def step(k, b):
            S_v, T_v, sc_v, dc_v, si, sS, sT, ex_v, dscat_v, ssem = bufs[b]
            Sn, Tn, scn, dcn, sin, sSn, sTn, exn, dsn, ssn = bufs[1 - b]
            rows_wait(S_v, T_v, sc_v, dc_v, sS, sT)
            for g in range(CH // 16):
                sl16 = pl.ds(g * 16, 16)
                dscat_v[sl16] = dc_v[sl16]
            idx_issue(k + 2, sc_v, dc_v, si)
            idx_wait(k + 1, scn, dcn, sin)

            @pl.when(k >= 1)
            def _():
                scat_wait(Sn, exn, dsn, ssn)

            rows_issue(Sn, Tn, scn, dcn, sSn, sTn)
            compute(S_v, T_v, ex_v, dscat_v, ssem)

        idx_wait(0, sc0_v, dc0_v, si0)
        rows_issue(S0_v, T0_v, sc0_v, dc0_v, gsS0, gsT0)

        def pair(p, _):
            step(2 * p, 0)
            step(2 * p + 1, 1)
            return _
        lax.fori_loop(0, (NCH - 1) // 2, pair, 0)

        rows_wait(S0_v, T0_v, sc0_v, dc0_v, gsS0, gsT0)
        for g in range(CH // 16):
            sl16 = pl.ds(g * 16, 16)
            ds0_v[sl16] = dc0_v[sl16]
        compute(S0_v, T0_v, ex0_v, ds0_v, ss0)
        scat_wait(S1_v, ex1_v, ds1_v, ss1)
        scat_wait(S0_v, ex0_v, ds0_v, ss0)
        idx_wait(NCH, sc1_v, dc1_v, si1)

        plsc.subcore_barrier()

        for j in range(ROWS_PER_TILE // CH):
            r0 = sid * ROWS_PER_TILE + j * CH
            pltpu.sync_copy(acc_s.at[pl.ds(r0, CH)], S0_v)
            pltpu.sync_copy(S0_v, outp_h.at[cid, pl.ds(r0, CH)])
        for j in range(ROWS_PER_TILE // CH):
            r0 = sid * ROWS_PER_TILE + j * CH
            pltpu.sync_copy(den_s.at[pl.ds(r0, CH)], ex0_v)
            pltpu.sync_copy(ex0_v, outd_h.at[pl.ds(cid * NPAD + r0, CH)])

    mesh = plsc.VectorSubcoreMesh(core_axis_name="c", subcore_axis_name="s")
    f = pl.kernel(
        body,
        out_type=[
            jax.ShapeDtypeStruct((2, NPAD, D), jnp.float32),
            jax.ShapeDtypeStruct((2 * NPAD,), jnp.float32),
        ],
        mesh=mesh,
        compiler_params=pltpu.CompilerParams(needs_layout_passes=False),
        scratch_types=[
            pltpu.VMEM((CH,), jnp.int32),
            pltpu.VMEM((CH,), jnp.int32),
            pltpu.VMEM((CH,), jnp.int32),
            pltpu.VMEM((CH,), jnp.int32),
            pltpu.VMEM((CH,), jnp.int32),
            pltpu.VMEM((CH,), jnp.int32),
            pltpu.VMEM((CH, D), jnp.float32),
            pltpu.VMEM((CH, D), jnp.float32),
            pltpu.VMEM((CH, D), jnp.float32),
            pltpu.VMEM((CH, D), jnp.float32),
            pltpu.VMEM((CH,), jnp.float32),
            pltpu.VMEM((CH,), jnp.float32),
            pltpu.VMEM((16,), jnp.float32),
            pltpu.VMEM_SHARED((NPAD, D), jnp.float32),
            pltpu.VMEM_SHARED((NPAD,), jnp.float32),
            pltpu.SemaphoreType.DMA,
            pltpu.SemaphoreType.DMA,
            pltpu.SemaphoreType.DMA,
            pltpu.SemaphoreType.DMA,
            pltpu.SemaphoreType.DMA,
            pltpu.SemaphoreType.DMA,
            pltpu.SemaphoreType.DMA,
            pltpu.SemaphoreType.DMA,
        ],
    )
    acc, den = f(tab, src, dst, beta16)
    return acc, den.reshape(2, NPAD)


def _tc_proj(x, W):
    Din = W.shape[0]

    def tc_body(x_ref, w_ref, o_ref):
        o_ref[...] = jnp.dot(x_ref[...], w_ref[...],
                             preferred_element_type=jnp.float32)

    return pl.pallas_call(
        tc_body,
        grid=(NPAD // RB,),
        in_specs=[
            pl.BlockSpec((RB, Din), lambda i: (i, 0)),
            pl.BlockSpec((Din, D), lambda i: (0, 0)),
        ],
        out_specs=pl.BlockSpec((RB, D), lambda i: (i, 0)),
        out_shape=jax.ShapeDtypeStruct((NPAD, D), jnp.float32),
    )(x, W)


def _tc_combine_proj(p, dnm, W):
    C = W.shape[1]

    def tc_body(p0_ref, p1_ref, d0_ref, d1_ref, w_ref, o_ref):
        den = d0_ref[...] + d1_ref[...] + 1e-16
        z = jnp.maximum((p0_ref[...] + p1_ref[...]) / den[:, None], 0.0)
        h = jnp.dot(z, w_ref[...], preferred_element_type=jnp.float32)
        o_ref[...] = jnp.concatenate(
            [h, jnp.zeros((RB, D - C), jnp.float32)], axis=1)

    return pl.pallas_call(
        tc_body,
        grid=(NPAD // RB,),
        in_specs=[
            pl.BlockSpec((RB, D), lambda i: (i, 0)),
            pl.BlockSpec((RB, D), lambda i: (i, 0)),
            pl.BlockSpec((RB,), lambda i: (i,)),
            pl.BlockSpec((RB,), lambda i: (i,)),
            pl.BlockSpec((D, C), lambda i: (0, 0)),
        ],
        out_specs=pl.BlockSpec((RB, D), lambda i: (i, 0)),
        out_shape=jax.ShapeDtypeStruct((NPAD, D), jnp.float32),
    )(p[0], p[1], dnm[0], dnm[1], W)


def _tc_combine_logsoftmax(q, dnm, C):
    def tc_body(q0_ref, q1_ref, d0_ref, d1_ref, o_ref):
        den = d0_ref[...] + d1_ref[...] + 1e-16
        o = (q0_ref[...] + q1_ref[...])[:, :C] / den[:, None]
        m = jnp.max(o, axis=1, keepdims=True)
        ex = jnp.exp(o - m)
        lse = jnp.log(jnp.sum(ex, axis=1, keepdims=True))
        o_ref[...] = o - m - lse

    return pl.pallas_call(
        tc_body,
        grid=(NPAD // RB,),
        in_specs=[
            pl.BlockSpec((RB, D), lambda i: (i, 0)),
            pl.BlockSpec((RB, D), lambda i: (i, 0)),
            pl.BlockSpec((RB,), lambda i: (i,)),
            pl.BlockSpec((RB,), lambda i: (i,)),
        ],
        out_specs=pl.BlockSpec((RB, C), lambda i: (i, 0)),
        out_shape=jax.ShapeDtypeStruct((NPAD, C), jnp.float32),
    )(q[0], q[1], dnm[0], dnm[1])


@jax.jit
def kernel(x, edge_index, W1, beta1, W2, beta2):
    src = jnp.pad(edge_index[0], (0, 2 * CH))
    dst = jnp.pad(edge_index[1], (0, 2 * CH))
    x_pad = jnp.pad(x, ((0, NPAD - N), (0, 0)))

    tab1 = _tc_proj(x_pad, W1)
    b1 = jnp.full((16,), beta1, jnp.float32)
    p1, d1 = _sc_edge_layer(tab1, src, dst, b1)

    tab2 = _tc_combine_proj(p1, d1, W2)
    b2 = jnp.full((16,), beta2, jnp.float32)
    p2, d2 = _sc_edge_layer(tab2, src, dst, b2)

    out = _tc_combine_logsoftmax(p2, d2, W2.shape[1])
    return out[:N]


# speedup vs baseline: 25.4136x; 1.0130x over previous
import jax
import jax.numpy as jnp
from jax import lax
from jax.experimental import pallas as pl
from jax.experimental.pallas import tpu as pltpu
from jax.experimental.pallas import tpu_sc as plsc

N = 10000
E = 320000
D = 128
NPAD = 10240
ROWS_PER_TILE = NPAD // 16
CH = 80
EPT = E // 32
NCH = EPT // CH
RB = 2048


def _rsqrt_nr(x):
    i = plsc.bitcast(x, jnp.int32)
    i = jnp.int32(0x5F3759DF) - lax.shift_right_logical(i, 1)
    y = plsc.bitcast(i, jnp.float32)
    half = x * 0.5
    for _ in range(3):
        y = y * (1.5 - half * y * y)
    return y


def _sc_edge_layer(tabh, tabn, src, dst, beta16):
    def body(tabh_h, tabn_h, src_h, dst_h, beta_h, outp_h, outd_h,
             sc0_v, dc0_v, sc1_v, dc1_v, ds0_v, ds1_v,
             S0_v, T0_v, S1_v, T1_v, ex0_v, ex1_v, bv,
             acc_s, den_s, si0, si1, gsS0, gsT0, gsS1, gsT1, ss0, ss1):
        cid = lax.axis_index("c")
        sid = lax.axis_index("s")
        ebase = (cid * 16 + sid) * EPT

        pltpu.sync_copy(beta_h, bv)

        def idx_issue(k, sc_v, dc_v, sem):
            off = ebase + k * CH
            pltpu.async_copy(src_h.at[pl.ds(off, CH)], sc_v, sem)
            pltpu.async_copy(dst_h.at[pl.ds(off, CH)], dc_v, sem)

        def idx_wait(k, sc_v, dc_v, sem):
            off = ebase + k * CH
            pltpu.make_async_copy(src_h.at[pl.ds(off, CH)], sc_v, sem).wait()
            pltpu.make_async_copy(dst_h.at[pl.ds(off, CH)], dc_v, sem).wait()

        idx_issue(0, sc0_v, dc0_v, si0)
        idx_issue(1, sc1_v, dc1_v, si1)

        def zs_row(i, _):
            for j in range(D // 16):
                S0_v[i, pl.ds(j * 16, 16)] = jnp.zeros((16,), jnp.float32)
            return _
        lax.fori_loop(0, CH, zs_row, 0)
        for g in range(CH // 16):
            ex0_v[pl.ds(g * 16, 16)] = jnp.zeros((16,), jnp.float32)

        for j in range(ROWS_PER_TILE // CH):
            r0 = sid * ROWS_PER_TILE + j * CH
            pltpu.sync_copy(S0_v, acc_s.at[pl.ds(r0, CH)])
            pltpu.sync_copy(ex0_v, den_s.at[pl.ds(r0, CH)])

        plsc.subcore_barrier()

        bvec = bv[...]
        iota16 = lax.broadcasted_iota(jnp.int32, (16,), 0)

        def compute(S_v, T_v, ex_v, dscat_v, ssem):
            def group(g, _):
                sl = pl.ds(g * 16, 16)
                st = jnp.zeros((16,), jnp.float32)
                ss = jnp.zeros((16,), jnp.float32)
                for l in range(16):
                    e = g * 16 + l
                    s0 = S_v[e, pl.ds(0, 16)]
                    t0 = T_v[e, pl.ds(0, 16)]
                    a = s0 * t0
                    b = s0 * s0
                    for j in range(1, D // 16):
                        dsl = pl.ds(j * 16, 16)
                        sj = S_v[e, dsl]
                        tj = T_v[e, dsl]
                        a = a + sj * tj
                        b = b + sj * sj
                    lm = iota16 == l
                    st = jnp.where(lm, jnp.sum(a), st)
                    ss = jnp.where(lm, jnp.sum(b), ss)
                cos = st * _rsqrt_nr(jnp.maximum(ss, 1e-30))
                ex = jnp.exp(cos * bvec)
                ex_v[sl] = ex
                for l in range(16):
                    e = g * 16 + l
                    mm = ex[l]
                    for j in range(D // 16):
                        dsl = pl.ds(j * 16, 16)
                        S_v[e, dsl] = S_v[e, dsl] * mm
                return _
            lax.fori_loop(0, CH // 16, group, 0)

            pltpu.async_copy(ex_v, den_s.at[dscat_v], ssem, add=True)
            pltpu.async_copy(S_v, acc_s.at[dscat_v], ssem, add=True)

        def scat_wait(S_v, ex_v, dscat_v, ssem):
            pltpu.make_async_copy(ex_v, den_s.at[dscat_v], ssem).wait()
            pltpu.make_async_copy(S_v, acc_s.at[dscat_v], ssem).wait()

        def rows_issue(S_v, T_v, sc_v, dc_v, sS, sT):
            pltpu.async_copy(tabh_h.at[sc_v], S_v, sS)
            pltpu.async_copy(tabn_h.at[dc_v], T_v, sT)

        def rows_wait(S_v, T_v, sc_v, dc_v, sS, sT):
            pltpu.make_async_copy(tabh_h.at[sc_v], S_v, sS).wait()
            pltpu.make_async_copy(tabn_h.at[dc_v], T_v, sT).wait()

        bufs = [(S0_v, T0_v, sc0_v, dc0_v, si0, gsS0, gsT0, ex0_v, ds0_v, ss0),
                (S1_v, T1_v, sc1_v, dc1_v, si1, gsS1, gsT1, ex1_v, ds1_v, ss1)]

        def step(k, b):
            S_v, T_v, sc_v, dc_v, si, sS, sT, ex_v, dscat_v, ssem = bufs[b]
            Sn, Tn, scn, dcn, sin, sSn, sTn, exn, dsn, ssn = bufs[1 - b]
            rows_wait(S_v, T_v, sc_v, dc_v, sS, sT)
            for g in range(CH // 16):
                sl16 = pl.ds(g * 16, 16)
                dscat_v[sl16] = dc_v[sl16]
            idx_issue(k + 2, sc_v, dc_v, si)
            idx_wait(k + 1, scn, dcn, sin)

            @pl.when(k >= 1)
            def _():
                scat_wait(Sn, exn, dsn, ssn)

            rows_issue(Sn, Tn, scn, dcn, sSn, sTn)
            compute(S_v, T_v, ex_v, dscat_v, ssem)

        idx_wait(0, sc0_v, dc0_v, si0)
        rows_issue(S0_v, T0_v, sc0_v, dc0_v, gsS0, gsT0)

        def pair(p, _):
            step(2 * p, 0)
            step(2 * p + 1, 1)
            return _
        lax.fori_loop(0, (NCH - 1) // 2, pair, 0)

        rows_wait(S0_v, T0_v, sc0_v, dc0_v, gsS0, gsT0)
        for g in range(CH // 16):
            sl16 = pl.ds(g * 16, 16)
            ds0_v[sl16] = dc0_v[sl16]
        compute(S0_v, T0_v, ex0_v, ds0_v, ss0)
        scat_wait(S1_v, ex1_v, ds1_v, ss1)
        scat_wait(S0_v, ex0_v, ds0_v, ss0)
        idx_wait(NCH, sc1_v, dc1_v, si1)

        plsc.subcore_barrier()

        for j in range(ROWS_PER_TILE // CH):
            r0 = sid * ROWS_PER_TILE + j * CH
            pltpu.sync_copy(acc_s.at[pl.ds(r0, CH)], S0_v)
            pltpu.sync_copy(S0_v, outp_h.at[cid, pl.ds(r0, CH)])
        for j in range(ROWS_PER_TILE // CH):
            r0 = sid * ROWS_PER_TILE + j * CH
            pltpu.sync_copy(den_s.at[pl.ds(r0, CH)], ex0_v)
            pltpu.sync_copy(ex0_v, outd_h.at[pl.ds(cid * NPAD + r0, CH)])

    mesh = plsc.VectorSubcoreMesh(core_axis_name="c", subcore_axis_name="s")
    f = pl.kernel(
        body,
        out_type=[
            jax.ShapeDtypeStruct((2, NPAD, D), jnp.float32),
            jax.ShapeDtypeStruct((2 * NPAD,), jnp.float32),
        ],
        mesh=mesh,
        compiler_params=pltpu.CompilerParams(needs_layout_passes=False),
        scratch_types=[
            pltpu.VMEM((CH,), jnp.int32),
            pltpu.VMEM((CH,), jnp.int32),
            pltpu.VMEM((CH,), jnp.int32),
            pltpu.VMEM((CH,), jnp.int32),
            pltpu.VMEM((CH,), jnp.int32),
            pltpu.VMEM((CH,), jnp.int32),
            pltpu.VMEM((CH, D), jnp.float32),
            pltpu.VMEM((CH, D), jnp.float32),
            pltpu.VMEM((CH, D), jnp.float32),
            pltpu.VMEM((CH, D), jnp.float32),
            pltpu.VMEM((CH,), jnp.float32),
            pltpu.VMEM((CH,), jnp.float32),
            pltpu.VMEM((16,), jnp.float32),
            pltpu.VMEM_SHARED((NPAD, D), jnp.float32),
            pltpu.VMEM_SHARED((NPAD,), jnp.float32),
            pltpu.SemaphoreType.DMA,
            pltpu.SemaphoreType.DMA,
            pltpu.SemaphoreType.DMA,
            pltpu.SemaphoreType.DMA,
            pltpu.SemaphoreType.DMA,
            pltpu.SemaphoreType.DMA,
            pltpu.SemaphoreType.DMA,
            pltpu.SemaphoreType.DMA,
        ],
    )
    acc, den = f(tabh, tabn, src, dst, beta16)
    return acc, den.reshape(2, NPAD)


def _tc_proj(x, W):
    Din = W.shape[0]

    def tc_body(x_ref, w_ref, o_ref, n_ref):
        h = jnp.dot(x_ref[...], w_ref[...], preferred_element_type=jnp.float32)
        o_ref[...] = h
        s = jnp.sqrt(jnp.sum(h * h, axis=1, keepdims=True)) + 1e-8
        n_ref[...] = h / s

    return pl.pallas_call(
        tc_body,
        grid=(NPAD // RB,),
        in_specs=[
            pl.BlockSpec((RB, Din), lambda i: (i, 0)),
            pl.BlockSpec((Din, D), lambda i: (0, 0)),
        ],
        out_specs=[
            pl.BlockSpec((RB, D), lambda i: (i, 0)),
            pl.BlockSpec((RB, D), lambda i: (i, 0)),
        ],
        out_shape=[
            jax.ShapeDtypeStruct((NPAD, D), jnp.float32),
            jax.ShapeDtypeStruct((NPAD, D), jnp.float32),
        ],
    )(x, W)


def _tc_combine_proj(p, dnm, W):
    C = W.shape[1]

    def tc_body(p0_ref, p1_ref, d0_ref, d1_ref, w_ref, o_ref, n_ref):
        den = d0_ref[...] + d1_ref[...] + 1e-16
        z = jnp.maximum((p0_ref[...] + p1_ref[...]) / den[:, None], 0.0)
        h = jnp.dot(z, w_ref[...], preferred_element_type=jnp.float32)
        hp = jnp.concatenate([h, jnp.zeros((RB, D - C), jnp.float32)], axis=1)
        o_ref[...] = hp
        s = jnp.sqrt(jnp.sum(h * h, axis=1, keepdims=True)) + 1e-8
        n_ref[...] = hp / s

    return pl.pallas_call(
        tc_body,
        grid=(NPAD // RB,),
        in_specs=[
            pl.BlockSpec((RB, D), lambda i: (i, 0)),
            pl.BlockSpec((RB, D), lambda i: (i, 0)),
            pl.BlockSpec((RB,), lambda i: (i,)),
            pl.BlockSpec((RB,), lambda i: (i,)),
            pl.BlockSpec((D, C), lambda i: (0, 0)),
        ],
        out_specs=[
            pl.BlockSpec((RB, D), lambda i: (i, 0)),
            pl.BlockSpec((RB, D), lambda i: (i, 0)),
        ],
        out_shape=[
            jax.ShapeDtypeStruct((NPAD, D), jnp.float32),
            jax.ShapeDtypeStruct((NPAD, D), jnp.float32),
        ],
    )(p[0], p[1], dnm[0], dnm[1], W)


def _tc_combine_logsoftmax(q, dnm, C):
    def tc_body(q0_ref, q1_ref, d0_ref, d1_ref, o_ref):
        den = d0_ref[...] + d1_ref[...] + 1e-16
        o = (q0_ref[...] + q1_ref[...]) / den[:, None]
        m = jnp.max(o, axis=1, keepdims=True)
        ex = jnp.exp(o - m)
        lse = jnp.log(jnp.sum(ex, axis=1, keepdims=True))
        o_ref[...] = o - m - lse

    return pl.pallas_call(
        tc_body,
        grid=(NPAD // RB,),
        in_specs=[
            pl.BlockSpec((RB, C), lambda i: (i, 0)),
            pl.BlockSpec((RB, C), lambda i: (i, 0)),
            pl.BlockSpec((RB,), lambda i: (i,)),
            pl.BlockSpec((RB,), lambda i: (i,)),
        ],
        out_specs=pl.BlockSpec((RB, C), lambda i: (i, 0)),
        out_shape=jax.ShapeDtypeStruct((NPAD, C), jnp.float32),
    )(q[0], q[1], dnm[0], dnm[1])


@jax.jit
def kernel(x, edge_index, W1, beta1, W2, beta2):
    src = jnp.pad(edge_index[0], (0, 2 * CH))
    dst = jnp.pad(edge_index[1], (0, 2 * CH))
    x_pad = jnp.pad(x, ((0, NPAD - N), (0, 0)))

    tab1, tab1n = _tc_proj(x_pad, W1)
    b1 = jnp.full((16,), beta1, jnp.float32)
    p1, d1 = _sc_edge_layer(tab1, tab1n, src, dst, b1)

    tab2, tab2n = _tc_combine_proj(p1, d1, W2)
    b2 = jnp.full((16,), beta2, jnp.float32)
    p2, d2 = _sc_edge_layer(tab2, tab2n, src, dst, b2)

    C = W2.shape[1]
    out = _tc_combine_logsoftmax(p2[:, :, :C], d2, C)
    return out[:N]


# speedup vs baseline: 26.9928x; 1.0621x over previous
import jax
import jax.numpy as jnp
from jax import lax
from jax.experimental import pallas as pl
from jax.experimental.pallas import tpu as pltpu
from jax.experimental.pallas import tpu_sc as plsc

N = 10000
E = 320000
D = 128
NPAD = 10240
ROWS_PER_TILE = NPAD // 16
CH = 80
EPT = E // 32
NCH = EPT // CH
RB = 2048


def _rsqrt_nr(x):
    i = plsc.bitcast(x, jnp.int32)
    i = jnp.int32(0x5F3759DF) - lax.shift_right_logical(i, 1)
    y = plsc.bitcast(i, jnp.float32)
    half = x * 0.5
    for _ in range(3):
        y = y * (1.5 - half * y * y)
    return y


def _sc_edge_layer(tabh, tabn, src, dst, beta16):
    def body(tabh_h, tabn_h, src_h, dst_h, beta_h, outp_h, outd_h,
             sc0_v, dc0_v, sc1_v, dc1_v, ds0_v, ds1_v,
             S0_v, T0_v, S1_v, T1_v, ex0_v, ex1_v, bv,
             acc_s, den_s, si0, si1, gsS0, gsT0, gsS1, gsT1, ss0, ss1):
        cid = lax.axis_index("c")
        sid = lax.axis_index("s")
        ebase = (cid * 16 + sid) * EPT

        pltpu.sync_copy(beta_h, bv)

        def idx_issue(k, sc_v, dc_v, sem):
            off = ebase + k * CH
            pltpu.async_copy(src_h.at[pl.ds(off, CH)], sc_v, sem)
            pltpu.async_copy(dst_h.at[pl.ds(off, CH)], dc_v, sem)

        def idx_wait(k, sc_v, dc_v, sem):
            off = ebase + k * CH
            pltpu.make_async_copy(src_h.at[pl.ds(off, CH)], sc_v, sem).wait()
            pltpu.make_async_copy(dst_h.at[pl.ds(off, CH)], dc_v, sem).wait()

        idx_issue(0, sc0_v, dc0_v, si0)
        idx_issue(1, sc1_v, dc1_v, si1)

        def zs_row(i, _):
            for j in range(D // 16):
                S0_v[i, pl.ds(j * 16, 16)] = jnp.zeros((16,), jnp.float32)
            return _
        lax.fori_loop(0, CH, zs_row, 0)
        for g in range(CH // 16):
            ex0_v[pl.ds(g * 16, 16)] = jnp.zeros((16,), jnp.float32)

        for j in range(ROWS_PER_TILE // CH):
            r0 = sid * ROWS_PER_TILE + j * CH
            pltpu.sync_copy(S0_v, acc_s.at[pl.ds(r0, CH)])
            pltpu.sync_copy(ex0_v, den_s.at[pl.ds(r0, CH)])

        plsc.subcore_barrier()

        bvec = bv[...]
        iota16 = lax.broadcasted_iota(jnp.int32, (16,), 0)

        def compute(S_v, T_v, ex_v, dscat_v, ssem):
            def group(g, _):
                sl = pl.ds(g * 16, 16)
                ex16 = jnp.zeros((16,), jnp.float32)
                for sb in range(8):
                    e0 = g * 16 + 2 * sb
                    e1 = e0 + 1
                    s0r = [S_v[e0, pl.ds(j * 16, 16)] for j in range(D // 16)]
                    s1r = [S_v[e1, pl.ds(j * 16, 16)] for j in range(D // 16)]
                    a0 = s0r[0] * T_v[e0, pl.ds(0, 16)]
                    b0 = s0r[0] * s0r[0]
                    a1 = s1r[0] * T_v[e1, pl.ds(0, 16)]
                    b1 = s1r[0] * s1r[0]
                    for j in range(1, D // 16):
                        dsl = pl.ds(j * 16, 16)
                        a0 = a0 + s0r[j] * T_v[e0, dsl]
                        b0 = b0 + s0r[j] * s0r[j]
                        a1 = a1 + s1r[j] * T_v[e1, dsl]
                        b1 = b1 + s1r[j] * s1r[j]
                    lm0 = iota16 == (2 * sb)
                    lm1 = iota16 == (2 * sb + 1)
                    stv = jnp.where(lm0, jnp.sum(a0), jnp.zeros((16,), jnp.float32))
                    stv = jnp.where(lm1, jnp.sum(a1), stv)
                    ssv = jnp.where(lm0, jnp.sum(b0), jnp.ones((16,), jnp.float32))
                    ssv = jnp.where(lm1, jnp.sum(b1), ssv)
                    cos = stv * _rsqrt_nr(jnp.maximum(ssv, 1e-30))
                    exv = jnp.exp(cos * bvec)
                    ex16 = jnp.where(lm0 | lm1, exv, ex16)
                    mm0 = exv[2 * sb]
                    mm1 = exv[2 * sb + 1]
                    for j in range(D // 16):
                        dsl = pl.ds(j * 16, 16)
                        S_v[e0, dsl] = s0r[j] * mm0
                        S_v[e1, dsl] = s1r[j] * mm1
                ex_v[sl] = ex16
                return _
            lax.fori_loop(0, CH // 16, group, 0)

            pltpu.async_copy(ex_v, den_s.at[dscat_v], ssem, add=True)
            pltpu.async_copy(S_v, acc_s.at[dscat_v], ssem, add=True)

        def scat_wait(S_v, ex_v, dscat_v, ssem):
            pltpu.make_async_copy(ex_v, den_s.at[dscat_v], ssem).wait()
            pltpu.make_async_copy(S_v, acc_s.at[dscat_v], ssem).wait()

        def rows_issue(S_v, T_v, sc_v, dc_v, sS, sT):
            pltpu.async_copy(tabh_h.at[sc_v], S_v, sS)
            pltpu.async_copy(tabn_h.at[dc_v], T_v, sT)

        def rows_wait(S_v, T_v, sc_v, dc_v, sS, sT):
            pltpu.make_async_copy(tabh_h.at[sc_v], S_v, sS).wait()
            pltpu.make_async_copy(tabn_h.at[dc_v], T_v, sT).wait()

        bufs = [(S0_v, T0_v, sc0_v, dc0_v, si0, gsS0, gsT0, ex0_v, ds0_v, ss0),
                (S1_v, T1_v, sc1_v, dc1_v, si1, gsS1, gsT1, ex1_v, ds1_v, ss1)]

        def step(k, b):
            S_v, T_v, sc_v, dc_v, si, sS, sT, ex_v, dscat_v, ssem = bufs[b]
            Sn, Tn, scn, dcn, sin, sSn, sTn, exn, dsn, ssn = bufs[1 - b]
            rows_wait(S_v, T_v, sc_v, dc_v, sS, sT)
            for g in range(CH // 16):
                sl16 = pl.ds(g * 16, 16)
                dscat_v[sl16] = dc_v[sl16]
            idx_issue(k + 2, sc_v, dc_v, si)
            idx_wait(k + 1, scn, dcn, sin)

            @pl.when(k >= 1)
            def _():
                scat_wait(Sn, exn, dsn, ssn)

            rows_issue(Sn, Tn, scn, dcn, sSn, sTn)
            compute(S_v, T_v, ex_v, dscat_v, ssem)

        idx_wait(0, sc0_v, dc0_v, si0)
        rows_issue(S0_v, T0_v, sc0_v, dc0_v, gsS0, gsT0)

        def pair(p, _):
            step(2 * p, 0)
            step(2 * p + 1, 1)
            return _
        lax.fori_loop(0, (NCH - 1) // 2, pair, 0)

        rows_wait(S0_v, T0_v, sc0_v, dc0_v, gsS0, gsT0)
        for g in range(CH // 16):
            sl16 = pl.ds(g * 16, 16)
            ds0_v[sl16] = dc0_v[sl16]
        compute(S0_v, T0_v, ex0_v, ds0_v, ss0)
        scat_wait(S1_v, ex1_v, ds1_v, ss1)
        scat_wait(S0_v, ex0_v, ds0_v, ss0)
        idx_wait(NCH, sc1_v, dc1_v, si1)

        plsc.subcore_barrier()

        for j in range(ROWS_PER_TILE // CH):
            r0 = sid * ROWS_PER_TILE + j * CH
            pltpu.sync_copy(acc_s.at[pl.ds(r0, CH)], S0_v)
            pltpu.sync_copy(S0_v, outp_h.at[cid, pl.ds(r0, CH)])
        for j in range(ROWS_PER_TILE // CH):
            r0 = sid * ROWS_PER_TILE + j * CH
            pltpu.sync_copy(den_s.at[pl.ds(r0, CH)], ex0_v)
            pltpu.sync_copy(ex0_v, outd_h.at[pl.ds(cid * NPAD + r0, CH)])

    mesh = plsc.VectorSubcoreMesh(core_axis_name="c", subcore_axis_name="s")
    f = pl.kernel(
        body,
        out_type=[
            jax.ShapeDtypeStruct((2, NPAD, D), jnp.float32),
            jax.ShapeDtypeStruct((2 * NPAD,), jnp.float32),
        ],
        mesh=mesh,
        compiler_params=pltpu.CompilerParams(needs_layout_passes=False),
        scratch_types=[
            pltpu.VMEM((CH,), jnp.int32),
            pltpu.VMEM((CH,), jnp.int32),
            pltpu.VMEM((CH,), jnp.int32),
            pltpu.VMEM((CH,), jnp.int32),
            pltpu.VMEM((CH,), jnp.int32),
            pltpu.VMEM((CH,), jnp.int32),
            pltpu.VMEM((CH, D), jnp.float32),
            pltpu.VMEM((CH, D), jnp.float32),
            pltpu.VMEM((CH, D), jnp.float32),
            pltpu.VMEM((CH, D), jnp.float32),
            pltpu.VMEM((CH,), jnp.float32),
            pltpu.VMEM((CH,), jnp.float32),
            pltpu.VMEM((16,), jnp.float32),
            pltpu.VMEM_SHARED((NPAD, D), jnp.float32),
            pltpu.VMEM_SHARED((NPAD,), jnp.float32),
            pltpu.SemaphoreType.DMA,
            pltpu.SemaphoreType.DMA,
            pltpu.SemaphoreType.DMA,
            pltpu.SemaphoreType.DMA,
            pltpu.SemaphoreType.DMA,
            pltpu.SemaphoreType.DMA,
            pltpu.SemaphoreType.DMA,
            pltpu.SemaphoreType.DMA,
        ],
    )
    acc, den = f(tabh, tabn, src, dst, beta16)
    return acc, den.reshape(2, NPAD)


def _tc_proj(x, W):
    Din = W.shape[0]

    def tc_body(x_ref, w_ref, o_ref, n_ref):
        h = jnp.dot(x_ref[...], w_ref[...], preferred_element_type=jnp.float32)
        o_ref[...] = h
        s = jnp.sqrt(jnp.sum(h * h, axis=1, keepdims=True)) + 1e-8
        n_ref[...] = h / s

    return pl.pallas_call(
        tc_body,
        grid=(NPAD // RB,),
        in_specs=[
            pl.BlockSpec((RB, Din), lambda i: (i, 0)),
            pl.BlockSpec((Din, D), lambda i: (0, 0)),
        ],
        out_specs=[
            pl.BlockSpec((RB, D), lambda i: (i, 0)),
            pl.BlockSpec((RB, D), lambda i: (i, 0)),
        ],
        out_shape=[
            jax.ShapeDtypeStruct((NPAD, D), jnp.float32),
            jax.ShapeDtypeStruct((NPAD, D), jnp.float32),
        ],
    )(x, W)


def _tc_combine_proj(p, dnm, W):
    C = W.shape[1]

    def tc_body(p0_ref, p1_ref, d0_ref, d1_ref, w_ref, o_ref, n_ref):
        den = d0_ref[...] + d1_ref[...] + 1e-16
        z = jnp.maximum((p0_ref[...] + p1_ref[...]) / den[:, None], 0.0)
        h = jnp.dot(z, w_ref[...], preferred_element_type=jnp.float32)
        hp = jnp.concatenate([h, jnp.zeros((RB, D - C), jnp.float32)], axis=1)
        o_ref[...] = hp
        s = jnp.sqrt(jnp.sum(h * h, axis=1, keepdims=True)) + 1e-8
        n_ref[...] = hp / s

    return pl.pallas_call(
        tc_body,
        grid=(NPAD // RB,),
        in_specs=[
            pl.BlockSpec((RB, D), lambda i: (i, 0)),
            pl.BlockSpec((RB, D), lambda i: (i, 0)),
            pl.BlockSpec((RB,), lambda i: (i,)),
            pl.BlockSpec((RB,), lambda i: (i,)),
            pl.BlockSpec((D, C), lambda i: (0, 0)),
        ],
        out_specs=[
            pl.BlockSpec((RB, D), lambda i: (i, 0)),
            pl.BlockSpec((RB, D), lambda i: (i, 0)),
        ],
        out_shape=[
            jax.ShapeDtypeStruct((NPAD, D), jnp.float32),
            jax.ShapeDtypeStruct((NPAD, D), jnp.float32),
        ],
    )(p[0], p[1], dnm[0], dnm[1], W)


def _tc_combine_logsoftmax(q, dnm, C):
    def tc_body(q0_ref, q1_ref, d0_ref, d1_ref, o_ref):
        den = d0_ref[...] + d1_ref[...] + 1e-16
        o = (q0_ref[...] + q1_ref[...]) / den[:, None]
        m = jnp.max(o, axis=1, keepdims=True)
        ex = jnp.exp(o - m)
        lse = jnp.log(jnp.sum(ex, axis=1, keepdims=True))
        o_ref[...] = o - m - lse

    return pl.pallas_call(
        tc_body,
        grid=(NPAD // RB,),
        in_specs=[
            pl.BlockSpec((RB, C), lambda i: (i, 0)),
            pl.BlockSpec((RB, C), lambda i: (i, 0)),
            pl.BlockSpec((RB,), lambda i: (i,)),
            pl.BlockSpec((RB,), lambda i: (i,)),
        ],
        out_specs=pl.BlockSpec((RB, C), lambda i: (i, 0)),
        out_shape=jax.ShapeDtypeStruct((NPAD, C), jnp.float32),
    )(q[0], q[1], dnm[0], dnm[1])


@jax.jit
def kernel(x, edge_index, W1, beta1, W2, beta2):
    src = jnp.pad(edge_index[0], (0, 2 * CH))
    dst = jnp.pad(edge_index[1], (0, 2 * CH))
    x_pad = jnp.pad(x, ((0, NPAD - N), (0, 0)))

    tab1, tab1n = _tc_proj(x_pad, W1)
    b1 = jnp.full((16,), beta1, jnp.float32)
    p1, d1 = _sc_edge_layer(tab1, tab1n, src, dst, b1)

    tab2, tab2n = _tc_combine_proj(p1, d1, W2)
    b2 = jnp.full((16,), beta2, jnp.float32)
    p2, d2 = _sc_edge_layer(tab2, tab2n, src, dst, b2)

    C = W2.shape[1]
    out = _tc_combine_logsoftmax(p2[:, :, :C], d2, C)
    return out[:N]


# speedup vs baseline: 27.9766x; 1.0364x over previous
import jax
import jax.numpy as jnp
from jax import lax
from jax.experimental import pallas as pl
from jax.experimental.pallas import tpu as pltpu
from jax.experimental.pallas import tpu_sc as plsc

N = 10000
E = 320000
D = 128
NPAD = 10240
ROWS_PER_TILE = NPAD // 16
CH = 80
EPT = E // 32
NCH = EPT // CH
RB = 2048


def _rsqrt_nr(x):
    i = plsc.bitcast(x, jnp.int32)
    i = jnp.int32(0x5F3759DF) - lax.shift_right_logical(i, 1)
    y = plsc.bitcast(i, jnp.float32)
    half = x * 0.5
    for _ in range(3):
        y = y * (1.5 - half * y * y)
    return y


def _sc_edge_layer(tabh, tabn, src, dst, beta16, ndot=D // 16, toff=0):
    def body(tabh_h, tabn_h, src_h, dst_h, beta_h, outp_h, outd_h,
             sc0_v, dc0_v, sc1_v, dc1_v, ds0_v, ds1_v,
             S0_v, T0_v, S1_v, T1_v, ex0_v, ex1_v, bv,
             acc_s, den_s, si0, si1, gsS0, gsT0, gsS1, gsT1, ss0, ss1):
        cid = lax.axis_index("c")
        sid = lax.axis_index("s")
        ebase = (cid * 16 + sid) * EPT

        pltpu.sync_copy(beta_h, bv)

        def idx_issue(k, sc_v, dc_v, sem):
            off = ebase + k * CH
            pltpu.async_copy(src_h.at[pl.ds(off, CH)], sc_v, sem)
            pltpu.async_copy(dst_h.at[pl.ds(off, CH)], dc_v, sem)

        def idx_wait(k, sc_v, dc_v, sem):
            off = ebase + k * CH
            pltpu.make_async_copy(src_h.at[pl.ds(off, CH)], sc_v, sem).wait()
            pltpu.make_async_copy(dst_h.at[pl.ds(off, CH)], dc_v, sem).wait()

        idx_issue(0, sc0_v, dc0_v, si0)
        idx_issue(1, sc1_v, dc1_v, si1)

        def zs_row(i, _):
            for j in range(D // 16):
                S0_v[i, pl.ds(j * 16, 16)] = jnp.zeros((16,), jnp.float32)
            return _
        lax.fori_loop(0, CH, zs_row, 0)
        for g in range(CH // 16):
            ex0_v[pl.ds(g * 16, 16)] = jnp.zeros((16,), jnp.float32)

        for j in range(ROWS_PER_TILE // CH):
            r0 = sid * ROWS_PER_TILE + j * CH
            pltpu.async_copy(S0_v, acc_s.at[pl.ds(r0, CH)], ss0)
            pltpu.async_copy(ex0_v, den_s.at[pl.ds(r0, CH)], ss0)
        for j in range(ROWS_PER_TILE // CH):
            r0 = sid * ROWS_PER_TILE + j * CH
            pltpu.make_async_copy(S0_v, acc_s.at[pl.ds(r0, CH)], ss0).wait()
            pltpu.make_async_copy(ex0_v, den_s.at[pl.ds(r0, CH)], ss0).wait()

        plsc.subcore_barrier()

        bvec = bv[...]
        iota16 = lax.broadcasted_iota(jnp.int32, (16,), 0)

        def compute(S_v, T_v, ex_v, dscat_v, ssem):
            def group(g, _):
                sl = pl.ds(g * 16, 16)
                ex16 = jnp.zeros((16,), jnp.float32)
                for sb in range(8):
                    e0 = g * 16 + 2 * sb
                    e1 = e0 + 1
                    s0r = [S_v[e0, pl.ds(j * 16, 16)] for j in range(ndot)]
                    s1r = [S_v[e1, pl.ds(j * 16, 16)] for j in range(ndot)]
                    a0 = s0r[0] * T_v[e0, pl.ds(toff, 16)]
                    b0 = s0r[0] * s0r[0]
                    a1 = s1r[0] * T_v[e1, pl.ds(toff, 16)]
                    b1 = s1r[0] * s1r[0]
                    for j in range(1, ndot):
                        dsl = pl.ds(j * 16, 16)
                        tsl = pl.ds(toff + j * 16, 16)
                        a0 = a0 + s0r[j] * T_v[e0, tsl]
                        b0 = b0 + s0r[j] * s0r[j]
                        a1 = a1 + s1r[j] * T_v[e1, tsl]
                        b1 = b1 + s1r[j] * s1r[j]
                    lm0 = iota16 == (2 * sb)
                    lm1 = iota16 == (2 * sb + 1)
                    stv = jnp.where(lm0, jnp.sum(a0), jnp.zeros((16,), jnp.float32))
                    stv = jnp.where(lm1, jnp.sum(a1), stv)
                    ssv = jnp.where(lm0, jnp.sum(b0), jnp.ones((16,), jnp.float32))
                    ssv = jnp.where(lm1, jnp.sum(b1), ssv)
                    cos = stv * _rsqrt_nr(jnp.maximum(ssv, 1e-30))
                    exv = jnp.exp(cos * bvec)
                    ex16 = jnp.where(lm0 | lm1, exv, ex16)
                    mm0 = exv[2 * sb]
                    mm1 = exv[2 * sb + 1]
                    for j in range(ndot):
                        dsl = pl.ds(j * 16, 16)
                        S_v[e0, dsl] = s0r[j] * mm0
                        S_v[e1, dsl] = s1r[j] * mm1
                ex_v[sl] = ex16
                return _
            lax.fori_loop(0, CH // 16, group, 0)

            pltpu.async_copy(ex_v, den_s.at[dscat_v], ssem, add=True)
            pltpu.async_copy(S_v, acc_s.at[dscat_v], ssem, add=True)

        def scat_wait(S_v, ex_v, dscat_v, ssem):
            pltpu.make_async_copy(ex_v, den_s.at[dscat_v], ssem).wait()
            pltpu.make_async_copy(S_v, acc_s.at[dscat_v], ssem).wait()

        def rows_issue(S_v, T_v, sc_v, dc_v, sS, sT):
            pltpu.async_copy(tabh_h.at[sc_v], S_v, sS)
            pltpu.async_copy(tabn_h.at[dc_v], T_v, sT)

        def rows_wait(S_v, T_v, sc_v, dc_v, sS, sT):
            pltpu.make_async_copy(tabh_h.at[sc_v], S_v, sS).wait()
            pltpu.make_async_copy(tabn_h.at[dc_v], T_v, sT).wait()

        bufs = [(S0_v, T0_v, sc0_v, dc0_v, si0, gsS0, gsT0, ex0_v, ds0_v, ss0),
                (S1_v, T1_v, sc1_v, dc1_v, si1, gsS1, gsT1, ex1_v, ds1_v, ss1)]

        def step(k, b):
            S_v, T_v, sc_v, dc_v, si, sS, sT, ex_v, dscat_v, ssem = bufs[b]
            Sn, Tn, scn, dcn, sin, sSn, sTn, exn, dsn, ssn = bufs[1 - b]
            rows_wait(S_v, T_v, sc_v, dc_v, sS, sT)
            for g in range(CH // 16):
                sl16 = pl.ds(g * 16, 16)
                dscat_v[sl16] = dc_v[sl16]
            idx_issue(k + 2, sc_v, dc_v, si)
            idx_wait(k + 1, scn, dcn, sin)

            @pl.when(k >= 1)
            def _():
                scat_wait(Sn, exn, dsn, ssn)

            rows_issue(Sn, Tn, scn, dcn, sSn, sTn)
            compute(S_v, T_v, ex_v, dscat_v, ssem)

        idx_wait(0, sc0_v, dc0_v, si0)
        rows_issue(S0_v, T0_v, sc0_v, dc0_v, gsS0, gsT0)

        def pair(p, _):
            step(2 * p, 0)
            step(2 * p + 1, 1)
            return _
        lax.fori_loop(0, (NCH - 1) // 2, pair, 0)

        rows_wait(S0_v, T0_v, sc0_v, dc0_v, gsS0, gsT0)
        for g in range(CH // 16):
            sl16 = pl.ds(g * 16, 16)
            ds0_v[sl16] = dc0_v[sl16]
        compute(S0_v, T0_v, ex0_v, ds0_v, ss0)
        scat_wait(S1_v, ex1_v, ds1_v, ss1)
        scat_wait(S0_v, ex0_v, ds0_v, ss0)
        idx_wait(NCH, sc1_v, dc1_v, si1)

        plsc.subcore_barrier()

        for j in range(ROWS_PER_TILE // CH):
            r0 = sid * ROWS_PER_TILE + j * CH
            pltpu.sync_copy(acc_s.at[pl.ds(r0, CH)], S0_v)
            pltpu.sync_copy(S0_v, outp_h.at[cid, pl.ds(r0, CH)])
        for j in range(ROWS_PER_TILE // CH):
            r0 = sid * ROWS_PER_TILE + j * CH
            pltpu.sync_copy(den_s.at[pl.ds(r0, CH)], ex0_v)
            pltpu.sync_copy(ex0_v, outd_h.at[pl.ds(cid * NPAD + r0, CH)])

    mesh = plsc.VectorSubcoreMesh(core_axis_name="c", subcore_axis_name="s")
    f = pl.kernel(
        body,
        out_type=[
            jax.ShapeDtypeStruct((2, NPAD, D), jnp.float32),
            jax.ShapeDtypeStruct((2 * NPAD,), jnp.float32),
        ],
        mesh=mesh,
        compiler_params=pltpu.CompilerParams(needs_layout_passes=False),
        scratch_types=[
            pltpu.VMEM((CH,), jnp.int32),
            pltpu.VMEM((CH,), jnp.int32),
            pltpu.VMEM((CH,), jnp.int32),
            pltpu.VMEM((CH,), jnp.int32),
            pltpu.VMEM((CH,), jnp.int32),
            pltpu.VMEM((CH,), jnp.int32),
            pltpu.VMEM((CH, D), jnp.float32),
            pltpu.VMEM((CH, D), jnp.float32),
            pltpu.VMEM((CH, D), jnp.float32),
            pltpu.VMEM((CH, D), jnp.float32),
            pltpu.VMEM((CH,), jnp.float32),
            pltpu.VMEM((CH,), jnp.float32),
            pltpu.VMEM((16,), jnp.float32),
            pltpu.VMEM_SHARED((NPAD, D), jnp.float32),
            pltpu.VMEM_SHARED((NPAD,), jnp.float32),
            pltpu.SemaphoreType.DMA,
            pltpu.SemaphoreType.DMA,
            pltpu.SemaphoreType.DMA,
            pltpu.SemaphoreType.DMA,
            pltpu.SemaphoreType.DMA,
            pltpu.SemaphoreType.DMA,
            pltpu.SemaphoreType.DMA,
            pltpu.SemaphoreType.DMA,
        ],
    )
    acc, den = f(tabh, tabn, src, dst, beta16)
    return acc, den.reshape(2, NPAD)


def _tc_proj(x, W):
    Din = W.shape[0]

    def tc_body(x_ref, w_ref, o_ref, n_ref):
        h = jnp.dot(x_ref[...], w_ref[...], preferred_element_type=jnp.float32)
        o_ref[...] = h
        s = jnp.sqrt(jnp.sum(h * h, axis=1, keepdims=True)) + 1e-8
        n_ref[...] = h / s

    return pl.pallas_call(
        tc_body,
        grid=(NPAD // RB,),
        in_specs=[
            pl.BlockSpec((RB, Din), lambda i: (i, 0)),
            pl.BlockSpec((Din, D), lambda i: (0, 0)),
        ],
        out_specs=[
            pl.BlockSpec((RB, D), lambda i: (i, 0)),
            pl.BlockSpec((RB, D), lambda i: (i, 0)),
        ],
        out_shape=[
            jax.ShapeDtypeStruct((NPAD, D), jnp.float32),
            jax.ShapeDtypeStruct((NPAD, D), jnp.float32),
        ],
    )(x, W)


def _tc_combine_proj(p, dnm, W):
    C = W.shape[1]

    def tc_body(p0_ref, p1_ref, d0_ref, d1_ref, w_ref, o_ref):
        den = d0_ref[...] + d1_ref[...] + 1e-16
        z = jnp.maximum((p0_ref[...] + p1_ref[...]) / den[:, None], 0.0)
        h = jnp.dot(z, w_ref[...], preferred_element_type=jnp.float32)
        s = jnp.sqrt(jnp.sum(h * h, axis=1, keepdims=True)) + 1e-8
        o_ref[...] = jnp.concatenate([h, h / s], axis=1)

    return pl.pallas_call(
        tc_body,
        grid=(NPAD // RB,),
        in_specs=[
            pl.BlockSpec((RB, D), lambda i: (i, 0)),
            pl.BlockSpec((RB, D), lambda i: (i, 0)),
            pl.BlockSpec((RB,), lambda i: (i,)),
            pl.BlockSpec((RB,), lambda i: (i,)),
            pl.BlockSpec((D, C), lambda i: (0, 0)),
        ],
        out_specs=pl.BlockSpec((RB, D), lambda i: (i, 0)),
        out_shape=jax.ShapeDtypeStruct((NPAD, D), jnp.float32),
    )(p[0], p[1], dnm[0], dnm[1], W)


def _tc_combine_logsoftmax(q, dnm, C):
    def tc_body(q0_ref, q1_ref, d0_ref, d1_ref, o_ref):
        den = d0_ref[...] + d1_ref[...] + 1e-16
        o = (q0_ref[...] + q1_ref[...])[:, :C] / den[:, None]
        m = jnp.max(o, axis=1, keepdims=True)
        ex = jnp.exp(o - m)
        lse = jnp.log(jnp.sum(ex, axis=1, keepdims=True))
        o_ref[...] = o - m - lse

    return pl.pallas_call(
        tc_body,
        grid=(NPAD // RB,),
        in_specs=[
            pl.BlockSpec((RB, D), lambda i: (i, 0)),
            pl.BlockSpec((RB, D), lambda i: (i, 0)),
            pl.BlockSpec((RB,), lambda i: (i,)),
            pl.BlockSpec((RB,), lambda i: (i,)),
        ],
        out_specs=pl.BlockSpec((RB, C), lambda i: (i, 0)),
        out_shape=jax.ShapeDtypeStruct((NPAD, C), jnp.float32),
    )(q[0], q[1], dnm[0], dnm[1])


@jax.jit
def kernel(x, edge_index, W1, beta1, W2, beta2):
    src = jnp.pad(edge_index[0], (0, 2 * CH))
    dst = jnp.pad(edge_index[1], (0, 2 * CH))
    x_pad = jnp.pad(x, ((0, NPAD - N), (0, 0)))

    tab1, tab1n = _tc_proj(x_pad, W1)
    b1 = jnp.full((16,), beta1, jnp.float32)
    p1, d1 = _sc_edge_layer(tab1, tab1n, src, dst, b1)

    tab2 = _tc_combine_proj(p1, d1, W2)
    b2 = jnp.full((16,), beta2, jnp.float32)
    C = W2.shape[1]
    p2, d2 = _sc_edge_layer(tab2, tab2, src, dst, b2, ndot=C // 16, toff=C)

    out = _tc_combine_logsoftmax(p2, d2, C)
    return out[:N]


# speedup vs baseline: 28.1240x; 1.0053x over previous
import jax
import jax.numpy as jnp
from jax import lax
from jax.experimental import pallas as pl
from jax.experimental.pallas import tpu as pltpu
from jax.experimental.pallas import tpu_sc as plsc

N = 10000
E = 320000
D = 128
NPAD = 10240
ROWS_PER_TILE = NPAD // 16
CH = 80
EPT = E // 32
NCH = EPT // CH
RB = 2048


def _rsqrt_nr(x):
    i = plsc.bitcast(x, jnp.int32)
    i = jnp.int32(0x5F3759DF) - lax.shift_right_logical(i, 1)
    y = plsc.bitcast(i, jnp.float32)
    half = x * 0.5
    for _ in range(3):
        y = y * (1.5 - half * y * y)
    return y


def _sc_edge_layer(tabh, tabn, src, dst, beta16, ndot=D // 16, toff=0):
    def body(tabh_h, tabn_h, src_h, dst_h, beta_h, outp_h, outd_h,
             sc0_v, dc0_v, sc1_v, dc1_v, ds0_v, ds1_v,
             S0_v, T0_v, S1_v, T1_v, ex0_v, ex1_v, bv,
             acc_s, den_s, si0, si1, gsS0, gsT0, gsS1, gsT1, ss0, ss1):
        cid = lax.axis_index("c")
        sid = lax.axis_index("s")
        ebase = (cid * 16 + sid) * EPT

        pltpu.sync_copy(beta_h, bv)

        def idx_issue(k, sc_v, dc_v, sem):
            off = jnp.minimum(ebase + k * CH, E - CH)
            pltpu.async_copy(src_h.at[pl.ds(off, CH)], sc_v, sem)
            pltpu.async_copy(dst_h.at[pl.ds(off, CH)], dc_v, sem)

        def idx_wait(k, sc_v, dc_v, sem):
            off = jnp.minimum(ebase + k * CH, E - CH)
            pltpu.make_async_copy(src_h.at[pl.ds(off, CH)], sc_v, sem).wait()
            pltpu.make_async_copy(dst_h.at[pl.ds(off, CH)], dc_v, sem).wait()

        idx_issue(0, sc0_v, dc0_v, si0)
        idx_issue(1, sc1_v, dc1_v, si1)
        idx_wait(0, sc0_v, dc0_v, si0)
        pltpu.async_copy(tabh_h.at[sc0_v], S0_v, gsS0)
        pltpu.async_copy(tabn_h.at[dc0_v], T0_v, gsT0)

        def zs_row(i, _):
            for j in range(D // 16):
                S1_v[i, pl.ds(j * 16, 16)] = jnp.zeros((16,), jnp.float32)
            return _
        lax.fori_loop(0, CH, zs_row, 0)
        for g in range(CH // 16):
            ex0_v[pl.ds(g * 16, 16)] = jnp.zeros((16,), jnp.float32)

        for j in range(ROWS_PER_TILE // CH):
            r0 = sid * ROWS_PER_TILE + j * CH
            pltpu.async_copy(S1_v, acc_s.at[pl.ds(r0, CH)], ss0)
            pltpu.async_copy(ex0_v, den_s.at[pl.ds(r0, CH)], ss0)
        for j in range(ROWS_PER_TILE // CH):
            r0 = sid * ROWS_PER_TILE + j * CH
            pltpu.make_async_copy(S1_v, acc_s.at[pl.ds(r0, CH)], ss0).wait()
            pltpu.make_async_copy(ex0_v, den_s.at[pl.ds(r0, CH)], ss0).wait()

        plsc.subcore_barrier()

        bvec = bv[...]
        iota16 = lax.broadcasted_iota(jnp.int32, (16,), 0)

        def compute(S_v, T_v, ex_v, dscat_v, ssem):
            def group(g, _):
                sl = pl.ds(g * 16, 16)
                ex16 = jnp.zeros((16,), jnp.float32)
                for sb in range(8):
                    e0 = g * 16 + 2 * sb
                    e1 = e0 + 1
                    s0r = [S_v[e0, pl.ds(j * 16, 16)] for j in range(ndot)]
                    s1r = [S_v[e1, pl.ds(j * 16, 16)] for j in range(ndot)]
                    a0 = s0r[0] * T_v[e0, pl.ds(toff, 16)]
                    b0 = s0r[0] * s0r[0]
                    a1 = s1r[0] * T_v[e1, pl.ds(toff, 16)]
                    b1 = s1r[0] * s1r[0]
                    for j in range(1, ndot):
                        dsl = pl.ds(j * 16, 16)
                        tsl = pl.ds(toff + j * 16, 16)
                        a0 = a0 + s0r[j] * T_v[e0, tsl]
                        b0 = b0 + s0r[j] * s0r[j]
                        a1 = a1 + s1r[j] * T_v[e1, tsl]
                        b1 = b1 + s1r[j] * s1r[j]
                    lm0 = iota16 == (2 * sb)
                    lm1 = iota16 == (2 * sb + 1)
                    stv = jnp.where(lm0, jnp.sum(a0), jnp.zeros((16,), jnp.float32))
                    stv = jnp.where(lm1, jnp.sum(a1), stv)
                    ssv = jnp.where(lm0, jnp.sum(b0), jnp.ones((16,), jnp.float32))
                    ssv = jnp.where(lm1, jnp.sum(b1), ssv)
                    cos = stv * _rsqrt_nr(jnp.maximum(ssv, 1e-30))
                    exv = jnp.exp(cos * bvec)
                    ex16 = jnp.where(lm0 | lm1, exv, ex16)
                    mm0 = exv[2 * sb]
                    mm1 = exv[2 * sb + 1]
                    for j in range(ndot):
                        dsl = pl.ds(j * 16, 16)
                        S_v[e0, dsl] = s0r[j] * mm0
                        S_v[e1, dsl] = s1r[j] * mm1
                ex_v[sl] = ex16
                return _
            lax.fori_loop(0, CH // 16, group, 0)

            pltpu.async_copy(ex_v, den_s.at[dscat_v], ssem, add=True)
            pltpu.async_copy(S_v, acc_s.at[dscat_v], ssem, add=True)

        def scat_wait(S_v, ex_v, dscat_v, ssem):
            pltpu.make_async_copy(ex_v, den_s.at[dscat_v], ssem).wait()
            pltpu.make_async_copy(S_v, acc_s.at[dscat_v], ssem).wait()

        def rows_issue(S_v, T_v, sc_v, dc_v, sS, sT):
            pltpu.async_copy(tabh_h.at[sc_v], S_v, sS)
            pltpu.async_copy(tabn_h.at[dc_v], T_v, sT)

        def rows_wait(S_v, T_v, sc_v, dc_v, sS, sT):
            pltpu.make_async_copy(tabh_h.at[sc_v], S_v, sS).wait()
            pltpu.make_async_copy(tabn_h.at[dc_v], T_v, sT).wait()

        bufs = [(S0_v, T0_v, sc0_v, dc0_v, si0, gsS0, gsT0, ex0_v, ds0_v, ss0),
                (S1_v, T1_v, sc1_v, dc1_v, si1, gsS1, gsT1, ex1_v, ds1_v, ss1)]

        def step(k, b):
            S_v, T_v, sc_v, dc_v, si, sS, sT, ex_v, dscat_v, ssem = bufs[b]
            Sn, Tn, scn, dcn, sin, sSn, sTn, exn, dsn, ssn = bufs[1 - b]
            rows_wait(S_v, T_v, sc_v, dc_v, sS, sT)
            for g in range(CH // 16):
                sl16 = pl.ds(g * 16, 16)
                dscat_v[sl16] = dc_v[sl16]
            idx_issue(k + 2, sc_v, dc_v, si)
            idx_wait(k + 1, scn, dcn, sin)

            @pl.when(k >= 1)
            def _():
                scat_wait(Sn, exn, dsn, ssn)

            rows_issue(Sn, Tn, scn, dcn, sSn, sTn)
            compute(S_v, T_v, ex_v, dscat_v, ssem)

        def pair(p, _):
            step(2 * p, 0)
            step(2 * p + 1, 1)
            return _
        lax.fori_loop(0, (NCH - 1) // 2, pair, 0)

        rows_wait(S0_v, T0_v, sc0_v, dc0_v, gsS0, gsT0)
        for g in range(CH // 16):
            sl16 = pl.ds(g * 16, 16)
            ds0_v[sl16] = dc0_v[sl16]
        compute(S0_v, T0_v, ex0_v, ds0_v, ss0)
        scat_wait(S1_v, ex1_v, ds1_v, ss1)
        scat_wait(S0_v, ex0_v, ds0_v, ss0)
        idx_wait(NCH, sc1_v, dc1_v, si1)

        plsc.subcore_barrier()

        for j in range(ROWS_PER_TILE // CH):
            r0 = sid * ROWS_PER_TILE + j * CH
            pltpu.sync_copy(acc_s.at[pl.ds(r0, CH)], S0_v)
            pltpu.sync_copy(S0_v, outp_h.at[cid, pl.ds(r0, CH)])
        for j in range(ROWS_PER_TILE // CH):
            r0 = sid * ROWS_PER_TILE + j * CH
            pltpu.sync_copy(den_s.at[pl.ds(r0, CH)], ex0_v)
            pltpu.sync_copy(ex0_v, outd_h.at[pl.ds(cid * NPAD + r0, CH)])

    mesh = plsc.VectorSubcoreMesh(core_axis_name="c", subcore_axis_name="s")
    f = pl.kernel(
        body,
        out_type=[
            jax.ShapeDtypeStruct((2, NPAD, D), jnp.float32),
            jax.ShapeDtypeStruct((2 * NPAD,), jnp.float32),
        ],
        mesh=mesh,
        compiler_params=pltpu.CompilerParams(needs_layout_passes=False),
        scratch_types=[
            pltpu.VMEM((CH,), jnp.int32),
            pltpu.VMEM((CH,), jnp.int32),
            pltpu.VMEM((CH,), jnp.int32),
            pltpu.VMEM((CH,), jnp.int32),
            pltpu.VMEM((CH,), jnp.int32),
            pltpu.VMEM((CH,), jnp.int32),
            pltpu.VMEM((CH, D), jnp.float32),
            pltpu.VMEM((CH, D), jnp.float32),
            pltpu.VMEM((CH, D), jnp.float32),
            pltpu.VMEM((CH, D), jnp.float32),
            pltpu.VMEM((CH,), jnp.float32),
            pltpu.VMEM((CH,), jnp.float32),
            pltpu.VMEM((16,), jnp.float32),
            pltpu.VMEM_SHARED((NPAD, D), jnp.float32),
            pltpu.VMEM_SHARED((NPAD,), jnp.float32),
            pltpu.SemaphoreType.DMA,
            pltpu.SemaphoreType.DMA,
            pltpu.SemaphoreType.DMA,
            pltpu.SemaphoreType.DMA,
            pltpu.SemaphoreType.DMA,
            pltpu.SemaphoreType.DMA,
            pltpu.SemaphoreType.DMA,
            pltpu.SemaphoreType.DMA,
        ],
    )
    acc, den = f(tabh, tabn, src, dst, beta16)
    return acc, den.reshape(2, NPAD)


def _tc_proj(x, W):
    Din = W.shape[0]

    def tc_body(x_ref, w_ref, o_ref, n_ref):
        h = jnp.dot(x_ref[...], w_ref[...], preferred_element_type=jnp.float32)
        o_ref[...] = h
        s = jnp.sqrt(jnp.sum(h * h, axis=1, keepdims=True)) + 1e-8
        n_ref[...] = h / s

    return pl.pallas_call(
        tc_body,
        grid=(NPAD // RB,),
        in_specs=[
            pl.BlockSpec((RB, Din), lambda i: (i, 0)),
            pl.BlockSpec((Din, D), lambda i: (0, 0)),
        ],
        out_specs=[
            pl.BlockSpec((RB, D), lambda i: (i, 0)),
            pl.BlockSpec((RB, D), lambda i: (i, 0)),
        ],
        out_shape=[
            jax.ShapeDtypeStruct((NPAD, D), jnp.float32),
            jax.ShapeDtypeStruct((NPAD, D), jnp.float32),
        ],
    )(x, W)


def _tc_combine_proj(p, dnm, W):
    C = W.shape[1]

    def tc_body(p0_ref, p1_ref, d0_ref, d1_ref, w_ref, o_ref):
        den = d0_ref[...] + d1_ref[...] + 1e-16
        z = jnp.maximum((p0_ref[...] + p1_ref[...]) / den[:, None], 0.0)
        h = jnp.dot(z, w_ref[...], preferred_element_type=jnp.float32)
        s = jnp.sqrt(jnp.sum(h * h, axis=1, keepdims=True)) + 1e-8
        o_ref[...] = jnp.concatenate([h, h / s], axis=1)

    return pl.pallas_call(
        tc_body,
        grid=(NPAD // RB,),
        in_specs=[
            pl.BlockSpec((RB, D), lambda i: (i, 0)),
            pl.BlockSpec((RB, D), lambda i: (i, 0)),
            pl.BlockSpec((RB,), lambda i: (i,)),
            pl.BlockSpec((RB,), lambda i: (i,)),
            pl.BlockSpec((D, C), lambda i: (0, 0)),
        ],
        out_specs=pl.BlockSpec((RB, D), lambda i: (i, 0)),
        out_shape=jax.ShapeDtypeStruct((NPAD, D), jnp.float32),
    )(p[0], p[1], dnm[0], dnm[1], W)


def _tc_combine_logsoftmax(q, dnm, C):
    def tc_body(q0_ref, q1_ref, d0_ref, d1_ref, o_ref):
        den = d0_ref[...] + d1_ref[...] + 1e-16
        o = (q0_ref[...] + q1_ref[...])[:, :C] / den[:, None]
        m = jnp.max(o, axis=1, keepdims=True)
        ex = jnp.exp(o - m)
        lse = jnp.log(jnp.sum(ex, axis=1, keepdims=True))
        o_ref[...] = o - m - lse

    return pl.pallas_call(
        tc_body,
        grid=(NPAD // RB,),
        in_specs=[
            pl.BlockSpec((RB, D), lambda i: (i, 0)),
            pl.BlockSpec((RB, D), lambda i: (i, 0)),
            pl.BlockSpec((RB,), lambda i: (i,)),
            pl.BlockSpec((RB,), lambda i: (i,)),
        ],
        out_specs=pl.BlockSpec((RB, C), lambda i: (i, 0)),
        out_shape=jax.ShapeDtypeStruct((NPAD, C), jnp.float32),
    )(q[0], q[1], dnm[0], dnm[1])


@jax.jit
def kernel(x, edge_index, W1, beta1, W2, beta2):
    src = edge_index[0]
    dst = edge_index[1]
    x_pad = jnp.pad(x, ((0, NPAD - N), (0, 0)))

    tab1, tab1n = _tc_proj(x_pad, W1)
    b1 = jnp.full((16,), beta1, jnp.float32)
    p1, d1 = _sc_edge_layer(tab1, tab1n, src, dst, b1)

    tab2 = _tc_combine_proj(p1, d1, W2)
    b2 = jnp.full((16,), beta2, jnp.float32)
    C = W2.shape[1]
    p2, d2 = _sc_edge_layer(tab2, tab2, src, dst, b2, ndot=C // 16, toff=C)

    out = _tc_combine_logsoftmax(p2, d2, C)
    return out[:N]
